# baseline (speedup 1.0000x reference)
import jax
import jax.numpy as jnp
from jax import lax
from jax.experimental import pallas as pl
from jax.experimental.pallas import tpu as pltpu
from jax.experimental.pallas import tpu_sc as plsc

BATCH = 4096
SEQ = 50
SEQ_PAD = 64
DIM = 32
NW = 32
ROWS_PER_W = BATCH // NW

VOCAB = 1000000
NTILE = 7813
RM_ROWS = NTILE * 32
KMAX = (NTILE + NW - 1) // NW

CBW = 32
NCW = ROWS_PER_W // CBW
SLW = CBW * SEQ

CB = 16
NCHUNK = ROWS_PER_W // CB
SLOTS = CB * SEQ


def _rbody(tokt_hbm, rm_hbm, in_v, st_v, sem_i, sem_o):
    cid = lax.axis_index("c")
    sid = lax.axis_index("s")
    wid = sid * 2 + cid
    lane = lax.iota(jnp.int32, 16)

    def col_body(k, _):
        c = k * NW + wid

        @pl.when(c < NTILE)
        def _():
            for g8 in range(4):
                pltpu.async_copy(
                    tokt_hbm.at[pl.ds(g8 * 8, 8), pl.ds(c * 128, 128)],
                    in_v.at[pl.ds(g8 * 8, 8)], sem_i)
            pltpu.make_async_copy(
                tokt_hbm.at[pl.ds(0, 32), pl.ds(0, 128)], in_v, sem_i).wait()

            def q_body(q, _):
                for kk in range(8):
                    rows = 16 * (kk & 1) + lane
                    col = jnp.broadcast_to(4 * q + kk // 2, (16,))
                    st_v[q, pl.ds(kk * 16, 16)] = plsc.load_gather(
                        in_v, [rows, col])
                return 0

            lax.fori_loop(0, 32, q_body, 0)
            pltpu.async_copy(st_v, rm_hbm.at[pl.ds(c * 32, 32)], sem_o).wait()

        return 0

    lax.fori_loop(0, KMAX, col_body, 0)


def _wbody(doc_hbm, wt_hbm, wout_hbm, idx_v, ridx_v, w32_v, wout_v, sem):
    cid = lax.axis_index("c")
    sid = lax.axis_index("s")
    wid = sid * 2 + cid
    lane = lax.iota(jnp.int32, 16)

    for chunk in range(NCW):
        g = wid * NCW + chunk
        pltpu.sync_copy(doc_hbm.at[g], idx_v)

        def shift_body(k, _):
            base = pl.multiple_of(k * 16, 16)
            ridx_v[pl.ds(base, 16)] = jnp.right_shift(idx_v[pl.ds(base, 16)], 5)
            return 0

        lax.fori_loop(0, SLW // 16, shift_body, 0)
        pltpu.async_copy(wt_hbm.at[ridx_v], w32_v, sem).wait()

        def row_body(r, _):
            for k in range(4):
                slot = jnp.minimum(r * SEQ + k * 16 + lane, SLW - 1)
                orig = plsc.load_gather(idx_v, [slot])
                val = plsc.load_gather(w32_v, [slot, jnp.bitwise_and(orig, 31)])
                wout_v[r, pl.ds(k * 16, 16)] = val
            return 0

        lax.fori_loop(0, CBW, row_body, 0)
        pltpu.sync_copy(wout_v, wout_hbm.at[pl.ds(g * CBW, CBW)])


def _mbody(doc_hbm, wp_hbm, rm_hbm, out_hbm,
           idx_v, ridx_v, cb_v, tok_v, wv, out_v, sem):
    cid = lax.axis_index("c")
    sid = lax.axis_index("s")
    wid = sid * 2 + cid
    lane = lax.iota(jnp.int32, 16)

    for chunk in range(NCHUNK):
        g = wid * NCHUNK + chunk
        pltpu.sync_copy(doc_hbm.at[g], idx_v)
        pltpu.sync_copy(wp_hbm.at[pl.ds(g * CB, CB)], wv)

        def prep_body(k, _):
            base = pl.multiple_of(k * 16, 16)
            t = idx_v[pl.ds(base, 16)]
            ridx_v[pl.ds(base, 16)] = jnp.right_shift(t, 2)
            cb_v[pl.ds(base, 16)] = jnp.bitwise_and(t, 3) * DIM
            return 0

        lax.fori_loop(0, SLOTS // 16, prep_body, 0)
        pltpu.async_copy(rm_hbm.at[ridx_v], tok_v, sem).wait()

        def row_body(r, _):
            w0 = wv[r, pl.ds(0, 16)]
            w1 = wv[r, pl.ds(16, 16)]
            w2 = wv[r, pl.ds(32, 16)]
            w3 = wv[r, pl.ds(48, 16)]
            w3m = jnp.where(lane < (SEQ - 48), w3, -jnp.inf)
            m = jnp.max(jnp.maximum(jnp.maximum(w0, w1), jnp.maximum(w2, w3m)))
            e0 = jnp.exp(w0 - m)
            e1 = jnp.exp(w1 - m)
            e2 = jnp.exp(w2 - m)
            e3 = jnp.exp(w3m - m)
            s = jnp.sum(e0 + e1 + e2 + e3)
            inv = 1.0 / jnp.broadcast_to(s, (16,))
            cs = [e0 * inv, e1 * inv, e2 * inv, e3 * inv]

            base = r * SEQ
            a0 = jnp.zeros((16,), jnp.float32)
            a1 = jnp.zeros((16,), jnp.float32)
            for l in range(SEQ):
                c = cs[l // 16][l % 16]
                slot = jnp.broadcast_to(base + l, (16,))
                off = plsc.load_gather(cb_v, [slot]) + lane
                t0 = plsc.load_gather(tok_v, [slot, off])
                t1 = plsc.load_gather(tok_v, [slot, off + 16])
                a0 = a0 + c * t0
                a1 = a1 + c * t1
            out_v[r, pl.ds(0, 16)] = a0
            out_v[r, pl.ds(16, 16)] = a1
            return 0

        lax.fori_loop(0, CB, row_body, 0)
        pltpu.sync_copy(out_v, out_hbm.at[pl.ds(g * CB, CB)])


def kernel(document, token_table, weight_table):
    doc = document.astype(jnp.int32)
    mesh = plsc.VectorSubcoreMesh(core_axis_name="c", subcore_axis_name="s")
    params_lin = pltpu.CompilerParams(
        needs_layout_passes=False, use_tc_tiling_on_sc=False)
    params_tiled = pltpu.CompilerParams(
        needs_layout_passes=False, use_tc_tiling_on_sc=True)

    rfn = pl.kernel(
        _rbody,
        out_type=jax.ShapeDtypeStruct((RM_ROWS, 128), jnp.float32),
        mesh=mesh,
        compiler_params=params_tiled,
        scratch_types=[
            pltpu.VMEM((32, 128), jnp.float32),
            pltpu.VMEM((32, 128), jnp.float32),
            pltpu.SemaphoreType.DMA,
            pltpu.SemaphoreType.DMA,
        ],
    )
    tok_rm = rfn(token_table.T)

    wt32 = weight_table.reshape(31250, 32)
    wfn = pl.kernel(
        _wbody,
        out_type=jax.ShapeDtypeStruct((BATCH, SEQ_PAD), jnp.float32),
        mesh=mesh,
        compiler_params=params_lin,
        scratch_types=[
            pltpu.VMEM((SLW,), jnp.int32),
            pltpu.VMEM((SLW,), jnp.int32),
            pltpu.VMEM((SLW, 32), jnp.float32),
            pltpu.VMEM((CBW, SEQ_PAD), jnp.float32),
            pltpu.SemaphoreType.DMA,
        ],
    )
    wpad = wfn(doc.reshape(NW * NCW, SLW), wt32)

    mfn = pl.kernel(
        _mbody,
        out_type=jax.ShapeDtypeStruct((BATCH, DIM), jnp.float32),
        mesh=mesh,
        compiler_params=params_tiled,
        scratch_types=[
            pltpu.VMEM((SLOTS,), jnp.int32),
            pltpu.VMEM((SLOTS,), jnp.int32),
            pltpu.VMEM((SLOTS,), jnp.int32),
            pltpu.VMEM((SLOTS, 128), jnp.float32),
            pltpu.VMEM((CB, SEQ_PAD), jnp.float32),
            pltpu.VMEM((CB, DIM), jnp.float32),
            pltpu.SemaphoreType.DMA,
        ],
    )
    return mfn(doc.reshape(NW * NCHUNK, SLOTS), wpad, tok_rm)


# speedup vs baseline: 1.2099x; 1.2099x over previous
import jax
import jax.numpy as jnp
from jax import lax
from jax.experimental import pallas as pl
from jax.experimental.pallas import tpu as pltpu
from jax.experimental.pallas import tpu_sc as plsc

BATCH = 4096
SEQ = 50
SEQ_PAD = 64
DIM = 32
NW = 32
ROWS_PER_W = BATCH // NW

VOCAB = 1000000
NTILE = 7813
RM_ROWS = NTILE * 32
KMAX = (NTILE + NW - 1) // NW

CBW = 32
NCW = ROWS_PER_W // CBW
SLW = CBW * SEQ

CB = 16
NCHUNK = ROWS_PER_W // CB
SLOTS = CB * SEQ


def _rbody(tokt_hbm, rm_hbm, in_v0, in_v1, st_v, sem_i0, sem_i1, sem_o):
    cid = lax.axis_index("c")
    sid = lax.axis_index("s")
    wid = sid * 2 + cid
    lane = lax.iota(jnp.int32, 16)
    in_bufs = (in_v0, in_v1)
    in_sems = (sem_i0, sem_i1)

    def issue_in(c, buf, sem):
        for g8 in range(4):
            pltpu.async_copy(
                tokt_hbm.at[pl.ds(g8 * 8, 8), pl.ds(c * 128, 128)],
                buf.at[pl.ds(g8 * 8, 8)], sem)

    issue_in(wid, in_v0, sem_i0)

    def col_body(kk, _):
        for b in range(2):
            k = 2 * kk + b
            c = k * NW + wid
            c_next = c + NW

            @pl.when(c < NTILE)
            def _():
                @pl.when(c_next < NTILE)
                def _():
                    issue_in(c_next, in_bufs[1 - b], in_sems[1 - b])

                in_v = in_bufs[b]
                pltpu.make_async_copy(
                    tokt_hbm.at[pl.ds(0, 32), pl.ds(0, 128)],
                    in_v, in_sems[b]).wait()

                def q_body(q, _):
                    for j in range(8):
                        rows = 16 * (j & 1) + lane
                        col = jnp.broadcast_to(4 * q + j // 2, (16,))
                        st_v[q, pl.ds(j * 16, 16)] = plsc.load_gather(
                            in_v, [rows, col])
                    return 0

                lax.fori_loop(0, 32, q_body, 0)
                pltpu.async_copy(
                    st_v, rm_hbm.at[pl.ds(c * 32, 32)], sem_o).wait()

        return 0

    lax.fori_loop(0, (KMAX + 1) // 2, col_body, 0)


def _wbody(doc_hbm, wt_hbm, wout_hbm, idx_v, ridx_v, w32_v, wout_v, sem):
    cid = lax.axis_index("c")
    sid = lax.axis_index("s")
    wid = sid * 2 + cid
    lane = lax.iota(jnp.int32, 16)

    for chunk in range(NCW):
        g = wid * NCW + chunk
        pltpu.sync_copy(doc_hbm.at[g], idx_v)

        def shift_body(k, _):
            base = pl.multiple_of(k * 16, 16)
            ridx_v[pl.ds(base, 16)] = jnp.right_shift(idx_v[pl.ds(base, 16)], 5)
            return 0

        lax.fori_loop(0, SLW // 16, shift_body, 0)
        pltpu.async_copy(wt_hbm.at[ridx_v], w32_v, sem).wait()

        def row_body(r, _):
            for k in range(4):
                slot = jnp.minimum(r * SEQ + k * 16 + lane, SLW - 1)
                orig = plsc.load_gather(idx_v, [slot])
                val = plsc.load_gather(w32_v, [slot, jnp.bitwise_and(orig, 31)])
                wout_v[r, pl.ds(k * 16, 16)] = val
            return 0

        lax.fori_loop(0, CBW, row_body, 0)
        pltpu.sync_copy(wout_v, wout_hbm.at[pl.ds(g * CBW, CBW)])


def _mbody(doc_hbm, wp_hbm, rm_hbm, out_hbm,
           idx_v, ridx_v, cb_v, tok_v, wv, out_v, sem):
    cid = lax.axis_index("c")
    sid = lax.axis_index("s")
    wid = sid * 2 + cid
    lane = lax.iota(jnp.int32, 16)

    for chunk in range(NCHUNK):
        g = wid * NCHUNK + chunk
        pltpu.sync_copy(doc_hbm.at[g], idx_v)
        pltpu.sync_copy(wp_hbm.at[pl.ds(g * CB, CB)], wv)

        def prep_body(k, _):
            base = pl.multiple_of(k * 16, 16)
            t = idx_v[pl.ds(base, 16)]
            ridx_v[pl.ds(base, 16)] = jnp.right_shift(t, 2)
            cb_v[pl.ds(base, 16)] = jnp.bitwise_and(t, 3) * DIM
            return 0

        lax.fori_loop(0, SLOTS // 16, prep_body, 0)
        pltpu.async_copy(rm_hbm.at[ridx_v], tok_v, sem).wait()

        def row_body(r, _):
            w0 = wv[r, pl.ds(0, 16)]
            w1 = wv[r, pl.ds(16, 16)]
            w2 = wv[r, pl.ds(32, 16)]
            w3 = wv[r, pl.ds(48, 16)]
            w3m = jnp.where(lane < (SEQ - 48), w3, -jnp.inf)
            m = jnp.max(jnp.maximum(jnp.maximum(w0, w1), jnp.maximum(w2, w3m)))
            e0 = jnp.exp(w0 - m)
            e1 = jnp.exp(w1 - m)
            e2 = jnp.exp(w2 - m)
            e3 = jnp.exp(w3m - m)
            s = jnp.sum(e0 + e1 + e2 + e3)
            inv = 1.0 / jnp.broadcast_to(s, (16,))
            cs = [e0 * inv, e1 * inv, e2 * inv, e3 * inv]

            base = r * SEQ
            a0 = jnp.zeros((16,), jnp.float32)
            a1 = jnp.zeros((16,), jnp.float32)
            for l in range(SEQ):
                c = cs[l // 16][l % 16]
                slot = jnp.broadcast_to(base + l, (16,))
                off = plsc.load_gather(cb_v, [slot]) + lane
                t0 = plsc.load_gather(tok_v, [slot, off])
                t1 = plsc.load_gather(tok_v, [slot, off + 16])
                a0 = a0 + c * t0
                a1 = a1 + c * t1
            out_v[r, pl.ds(0, 16)] = a0
            out_v[r, pl.ds(16, 16)] = a1
            return 0

        lax.fori_loop(0, CB, row_body, 0)
        pltpu.sync_copy(out_v, out_hbm.at[pl.ds(g * CB, CB)])


def kernel(document, token_table, weight_table):
    doc = document.astype(jnp.int32)
    mesh = plsc.VectorSubcoreMesh(core_axis_name="c", subcore_axis_name="s")
    params_lin = pltpu.CompilerParams(
        needs_layout_passes=False, use_tc_tiling_on_sc=False)
    params_tiled = pltpu.CompilerParams(
        needs_layout_passes=False, use_tc_tiling_on_sc=True)

    rfn = pl.kernel(
        _rbody,
        out_type=jax.ShapeDtypeStruct((RM_ROWS, 128), jnp.float32),
        mesh=mesh,
        compiler_params=params_tiled,
        scratch_types=[
            pltpu.VMEM((32, 128), jnp.float32),
            pltpu.VMEM((32, 128), jnp.float32),
            pltpu.VMEM((32, 128), jnp.float32),
            pltpu.SemaphoreType.DMA,
            pltpu.SemaphoreType.DMA,
            pltpu.SemaphoreType.DMA,
        ],
    )
    tok_rm = rfn(token_table.T)

    wt32 = weight_table.reshape(31250, 32)
    wfn = pl.kernel(
        _wbody,
        out_type=jax.ShapeDtypeStruct((BATCH, SEQ_PAD), jnp.float32),
        mesh=mesh,
        compiler_params=params_lin,
        scratch_types=[
            pltpu.VMEM((SLW,), jnp.int32),
            pltpu.VMEM((SLW,), jnp.int32),
            pltpu.VMEM((SLW, 32), jnp.float32),
            pltpu.VMEM((CBW, SEQ_PAD), jnp.float32),
            pltpu.SemaphoreType.DMA,
        ],
    )
    wpad = wfn(doc.reshape(NW * NCW, SLW), wt32)

    mfn = pl.kernel(
        _mbody,
        out_type=jax.ShapeDtypeStruct((BATCH, DIM), jnp.float32),
        mesh=mesh,
        compiler_params=params_tiled,
        scratch_types=[
            pltpu.VMEM((SLOTS,), jnp.int32),
            pltpu.VMEM((SLOTS,), jnp.int32),
            pltpu.VMEM((SLOTS,), jnp.int32),
            pltpu.VMEM((SLOTS, 128), jnp.float32),
            pltpu.VMEM((CB, SEQ_PAD), jnp.float32),
            pltpu.VMEM((CB, DIM), jnp.float32),
            pltpu.SemaphoreType.DMA,
        ],
    )
    return mfn(doc.reshape(NW * NCHUNK, SLOTS), wpad, tok_rm)


# speedup vs baseline: 2.2173x; 1.8326x over previous
import jax
import jax.numpy as jnp
from jax import lax
from jax.experimental import pallas as pl
from jax.experimental.pallas import tpu as pltpu
from jax.experimental.pallas import tpu_sc as plsc

BATCH = 4096
SEQ = 50
SEQ_PAD = 64
DIM = 32
NW = 32
ROWS_PER_W = BATCH // NW

CBW = 32
NCW = ROWS_PER_W // CBW
SLW = CBW * SEQ

CB = 64
NCHUNK = ROWS_PER_W // CB
SLOTS = CB * SEQ


def _wbody(doc_hbm, wt_hbm, wout_hbm, idx_v, ridx_v, w32_v, wout_v, sem):
    cid = lax.axis_index("c")
    sid = lax.axis_index("s")
    wid = sid * 2 + cid
    lane = lax.iota(jnp.int32, 16)

    for chunk in range(NCW):
        g = wid * NCW + chunk
        pltpu.sync_copy(doc_hbm.at[g], idx_v)

        def shift_body(k, _):
            base = pl.multiple_of(k * 16, 16)
            ridx_v[pl.ds(base, 16)] = jnp.right_shift(idx_v[pl.ds(base, 16)], 5)
            return 0

        lax.fori_loop(0, SLW // 16, shift_body, 0)
        pltpu.async_copy(wt_hbm.at[ridx_v], w32_v, sem).wait()

        def row_body(r, _):
            for k in range(4):
                slot = jnp.minimum(r * SEQ + k * 16 + lane, SLW - 1)
                orig = plsc.load_gather(idx_v, [slot])
                val = plsc.load_gather(w32_v, [slot, jnp.bitwise_and(orig, 31)])
                wout_v[r, pl.ds(k * 16, 16)] = val
            return 0

        lax.fori_loop(0, CBW, row_body, 0)
        pltpu.sync_copy(wout_v, wout_hbm.at[pl.ds(g * CBW, CBW)])


def _mbody(doc_hbm, wp_hbm, tok_hbm, out_hbm, idx_v, tok_v, wv, out_v, sem):
    cid = lax.axis_index("c")
    sid = lax.axis_index("s")
    wid = sid * 2 + cid
    lane = lax.iota(jnp.int32, 16)

    for chunk in range(NCHUNK):
        g = wid * NCHUNK + chunk
        pltpu.sync_copy(doc_hbm.at[g], idx_v)
        pltpu.sync_copy(wp_hbm.at[pl.ds(g * CB, CB)], wv)
        pltpu.async_copy(tok_hbm.at[idx_v], tok_v, sem).wait()

        def row_body(r, _):
            w0 = wv[r, pl.ds(0, 16)]
            w1 = wv[r, pl.ds(16, 16)]
            w2 = wv[r, pl.ds(32, 16)]
            w3 = wv[r, pl.ds(48, 16)]
            w3m = jnp.where(lane < (SEQ - 48), w3, -jnp.inf)
            m = jnp.max(jnp.maximum(jnp.maximum(w0, w1), jnp.maximum(w2, w3m)))
            e0 = jnp.exp(w0 - m)
            e1 = jnp.exp(w1 - m)
            e2 = jnp.exp(w2 - m)
            e3 = jnp.exp(w3m - m)
            s = jnp.sum(e0 + e1 + e2 + e3)
            inv = 1.0 / jnp.broadcast_to(s, (16,))
            cs = [e0 * inv, e1 * inv, e2 * inv, e3 * inv]

            base = r * SEQ
            a0 = jnp.zeros((16,), jnp.float32)
            a1 = jnp.zeros((16,), jnp.float32)
            for l in range(SEQ):
                c = cs[l // 16][l % 16]
                row = base + l
                t0 = tok_v[row, pl.ds(0, 16)]
                t1 = tok_v[row, pl.ds(16, 16)]
                a0 = a0 + c * t0
                a1 = a1 + c * t1
            out_v[r, pl.ds(0, 16)] = a0
            out_v[r, pl.ds(16, 16)] = a1
            return 0

        lax.fori_loop(0, CB, row_body, 0)
        pltpu.sync_copy(out_v, out_hbm.at[pl.ds(g * CB, CB)])


def kernel(document, token_table, weight_table):
    doc = document.astype(jnp.int32)
    mesh = plsc.VectorSubcoreMesh(core_axis_name="c", subcore_axis_name="s")
    params = pltpu.CompilerParams(
        needs_layout_passes=False, use_tc_tiling_on_sc=False)

    wt32 = weight_table.reshape(31250, 32)
    wfn = pl.kernel(
        _wbody,
        out_type=jax.ShapeDtypeStruct((BATCH, SEQ_PAD), jnp.float32),
        mesh=mesh,
        compiler_params=params,
        scratch_types=[
            pltpu.VMEM((SLW,), jnp.int32),
            pltpu.VMEM((SLW,), jnp.int32),
            pltpu.VMEM((SLW, 32), jnp.float32),
            pltpu.VMEM((CBW, SEQ_PAD), jnp.float32),
            pltpu.SemaphoreType.DMA,
        ],
    )
    wpad = wfn(doc.reshape(NW * NCW, SLW), wt32)

    mfn = pl.kernel(
        _mbody,
        out_type=jax.ShapeDtypeStruct((BATCH, DIM), jnp.float32),
        mesh=mesh,
        compiler_params=params,
        scratch_types=[
            pltpu.VMEM((SLOTS,), jnp.int32),
            pltpu.VMEM((SLOTS, DIM), jnp.float32),
            pltpu.VMEM((CB, SEQ_PAD), jnp.float32),
            pltpu.VMEM((CB, DIM), jnp.float32),
            pltpu.SemaphoreType.DMA,
        ],
    )
    return mfn(doc.reshape(NW * NCHUNK, SLOTS), wpad, token_table)
